# baseline (speedup 1.0000x reference)
import functools

import jax
import jax.numpy as jnp
from jax import lax
from jax.experimental import pallas as pl
from jax.experimental.pallas import tpu as pltpu
from jax.experimental.pallas import tpu_sc as plsc

_K = 50
_ROWS = 128
_N = 32768
_NW = 32
_RPW = _ROWS // _NW
_G = 16
_NMAX = _N // _G
_INT_MIN = -(2**31)


def _skey(v):
    b = lax.bitcast_convert_type(v, jnp.int32)
    return jnp.where(b >= 0, b, b ^ jnp.int32(0x7FFFFFFF))


def _count_ge_blocks(ref, nblk, c):
    cvec = jnp.full((16,), c, dtype=jnp.int32)

    def body(g, cnt):
        for j in range(8):
            v = ref[pl.ds(g * 128 + j * 16, 16)]
            cnt = cnt + jnp.where(v >= cvec, jnp.int32(1), jnp.int32(0))
        return cnt

    cnt_v = lax.fori_loop(0, nblk, body, jnp.zeros((16,), jnp.int32))
    return jnp.sum(cnt_v)


def _search_blocks(ref, nblk, k):
    def it(i, p):
        c = p + (jnp.int32(1) << (jnp.int32(31) - i))
        cnt = _count_ge_blocks(ref, nblk, c)
        return jnp.where(cnt >= k, c, p)

    return lax.fori_loop(0, 32, it, jnp.int32(_INT_MIN))


def _sc_body(x_hbm, out_hbm, row_v, maxes_v, cand_v, thr_v):
    wid = lax.axis_index("s") * 2 + lax.axis_index("c")
    thr_v[...] = jnp.zeros((16,), jnp.float32)
    lane = jax.lax.broadcasted_iota(jnp.int32, (16,), 0)

    for r in range(_RPW):
        row = wid * _RPW + r
        pltpu.sync_copy(x_hbm.at[row], row_v)

        def groups(g, _):
            base = g * (_G * 16)
            acc = _skey(row_v[pl.ds(base, 16)])
            for j in range(1, _G):
                acc = jnp.maximum(acc, _skey(row_v[pl.ds(base + j * 16, 16)]))
            maxes_v[pl.ds(g * 16, 16)] = acc
            return 0

        lax.fori_loop(0, _NMAX // 16, groups, 0)

        t0 = _search_blocks(maxes_v, _NMAX // 128, jnp.int32(_K))
        t0v = jnp.full((16,), t0, dtype=jnp.int32)

        def collect(g, off):
            for j in range(8):
                v = _skey(row_v[pl.ds(g * 128 + j * 16, 16)])
                m = v >= t0v
                plsc.store_compressed(cand_v.at[pl.ds(off, 16)], v, mask=m)
                pc = plsc.all_reduce_population_count(m)
                off = off + jnp.max(pc)
            return off

        off = lax.fori_loop(0, _N // 128, collect, jnp.int32(0))

        for j in range(8):
            cand_v[pl.ds(off + j * 16, 16)] = jnp.full((16,), jnp.int32(_INT_MIN))

        nblk = (off + jnp.int32(127)) // jnp.int32(128)
        kv = _search_blocks(cand_v, nblk, jnp.int32(_K))

        kvv = jnp.full((16,), kv, dtype=jnp.int32)
        bv = jnp.where(kvv >= 0, kvv, kvv ^ jnp.int32(0x7FFFFFFF))
        tvv = lax.bitcast_convert_type(bv, jnp.float32)
        thr_v[...] = jnp.where(lane == r, tvv, thr_v[...])

    pltpu.sync_copy(thr_v, out_hbm.at[wid])


_sc_thresholds = functools.partial(
    pl.kernel,
    out_type=jax.ShapeDtypeStruct((_NW, 16), jnp.float32),
    mesh=plsc.VectorSubcoreMesh(core_axis_name="c", subcore_axis_name="s"),
    compiler_params=pltpu.CompilerParams(needs_layout_passes=False),
    scratch_types=[
        pltpu.VMEM((_N,), jnp.float32),
        pltpu.VMEM((_NMAX,), jnp.int32),
        pltpu.VMEM((_N + 128,), jnp.int32),
        pltpu.VMEM((16,), jnp.float32),
    ],
)(_sc_body)


def _mask_kernel(x_ref, t_ref, o_ref):
    o_ref[...] = (x_ref[...] >= t_ref[...]).astype(jnp.float32)


@jax.jit
def kernel(x):
    m, n = x.shape
    thr = _sc_thresholds(x)
    thr_col = thr[:, :_RPW].reshape(m, 1)
    r = 16
    return pl.pallas_call(
        _mask_kernel,
        out_shape=jax.ShapeDtypeStruct((m, n), jnp.float32),
        grid=(m // r,),
        in_specs=[
            pl.BlockSpec((r, n), lambda i: (i, 0)),
            pl.BlockSpec((r, 1), lambda i: (i, 0)),
        ],
        out_specs=pl.BlockSpec((r, n), lambda i: (i, 0)),
    )(x, thr_col)


# speedup vs baseline: 1.3572x; 1.3572x over previous
import functools

import jax
import jax.numpy as jnp
from jax import lax
from jax.experimental import pallas as pl
from jax.experimental.pallas import tpu as pltpu
from jax.experimental.pallas import tpu_sc as plsc

_K = 50
_ROWS = 128
_N = 32768
_NW = 32
_RPW = _ROWS // _NW
_G = 32
_NMAX = _N // _G
_INT_MIN = -(2**31)


def _skey(v):
    b = lax.bitcast_convert_type(v, jnp.int32)
    return jnp.where(b >= 0, b, b ^ jnp.int32(0x7FFFFFFF))


def _count_ge_blocks(ref, nblk, c):
    cvec = jnp.full((16,), c, dtype=jnp.int32)

    def body(g, cnt):
        for j in range(8):
            v = ref[pl.ds(g * 128 + j * 16, 16)]
            cnt = cnt + jnp.where(v >= cvec, jnp.int32(1), jnp.int32(0))
        return cnt

    cnt_v = lax.fori_loop(0, nblk, body, jnp.zeros((16,), jnp.int32))
    return jnp.sum(cnt_v)


def _search_blocks(ref, nblk, k):
    def it(i, p):
        c = p + (jnp.int32(1) << (jnp.int32(31) - i))
        cnt = _count_ge_blocks(ref, nblk, c)
        return jnp.where(cnt >= k, c, p)

    return lax.fori_loop(0, 32, it, jnp.int32(_INT_MIN))


def _key_to_f32(kvv):
    bv = jnp.where(kvv >= 0, kvv, kvv ^ jnp.int32(0x7FFFFFFF))
    return lax.bitcast_convert_type(bv, jnp.float32)


def _sc_row(row_v, maxes_v, cand_v):
    def groups(g, _):
        base = g * (_G * 16)
        acc = row_v[pl.ds(base, 16)]
        for j in range(1, _G):
            acc = jnp.maximum(acc, row_v[pl.ds(base + j * 16, 16)])
        maxes_v[pl.ds(g * 16, 16)] = _skey(acc)
        return 0

    lax.fori_loop(0, _NMAX // 16, groups, 0)

    t0 = _search_blocks(maxes_v, _NMAX // 128, jnp.int32(_K))
    t0f = _key_to_f32(jnp.full((16,), t0, dtype=jnp.int32))

    def collect(g, off):
        for j in range(8):
            v = row_v[pl.ds(g * 128 + j * 16, 16)]
            m = v >= t0f
            vb = lax.bitcast_convert_type(v, jnp.int32)
            plsc.store_compressed(cand_v.at[pl.ds(off, 16)], vb, mask=m)
            pc = plsc.all_reduce_population_count(m)
            off = off + pc[0]
        return off

    off = lax.fori_loop(0, _N // 128, collect, jnp.int32(0))

    sent = jnp.full((16,), jnp.int32(-1))
    for j in range(8):
        cand_v[pl.ds(off + j * 16, 16)] = sent

    nblk = (off + jnp.int32(127)) // jnp.int32(128)

    def conv(g, _):
        for j in range(8):
            s = pl.ds(g * 128 + j * 16, 16)
            b = cand_v[s]
            cand_v[s] = jnp.where(b >= 0, b, b ^ jnp.int32(0x7FFFFFFF))
        return 0

    lax.fori_loop(0, nblk, conv, 0)
    return _search_blocks(cand_v, nblk, jnp.int32(_K))


def _sc_body(x_hbm, out_hbm, row_a, row_b, maxes_v, cand_v, thr_v, sem):
    wid = lax.axis_index("s") * 2 + lax.axis_index("c")
    thr_v[...] = jnp.zeros((16,), jnp.float32)
    lane = jax.lax.broadcasted_iota(jnp.int32, (16,), 0)

    bufs = [row_a, row_b]
    pltpu.sync_copy(x_hbm.at[wid * _RPW], bufs[0])
    for r in range(_RPW):
        cp = None
        if r + 1 < _RPW:
            cp = pltpu.async_copy(
                x_hbm.at[wid * _RPW + r + 1], bufs[(r + 1) % 2], sem
            )
        kv = _sc_row(bufs[r % 2], maxes_v, cand_v)
        tvv = _key_to_f32(jnp.full((16,), kv, dtype=jnp.int32))
        thr_v[...] = jnp.where(lane == r, tvv, thr_v[...])
        if cp is not None:
            cp.wait()

    pltpu.sync_copy(thr_v, out_hbm.at[wid])


_sc_thresholds = functools.partial(
    pl.kernel,
    out_type=jax.ShapeDtypeStruct((_NW, 16), jnp.float32),
    mesh=plsc.VectorSubcoreMesh(core_axis_name="c", subcore_axis_name="s"),
    compiler_params=pltpu.CompilerParams(needs_layout_passes=False),
    scratch_types=[
        pltpu.VMEM((_N,), jnp.float32),
        pltpu.VMEM((_N,), jnp.float32),
        pltpu.VMEM((_NMAX,), jnp.int32),
        pltpu.VMEM((_N + 128,), jnp.int32),
        pltpu.VMEM((16,), jnp.float32),
        pltpu.SemaphoreType.DMA,
    ],
)(_sc_body)


def _mask_kernel(x_ref, t_ref, o_ref):
    o_ref[...] = (x_ref[...] >= t_ref[...]).astype(jnp.float32)


@jax.jit
def kernel(x):
    m, n = x.shape
    thr = _sc_thresholds(x)
    thr_col = thr[:, :_RPW].reshape(m, 1)
    r = 16
    return pl.pallas_call(
        _mask_kernel,
        out_shape=jax.ShapeDtypeStruct((m, n), jnp.float32),
        grid=(m // r,),
        in_specs=[
            pl.BlockSpec((r, n), lambda i: (i, 0)),
            pl.BlockSpec((r, 1), lambda i: (i, 0)),
        ],
        out_specs=pl.BlockSpec((r, n), lambda i: (i, 0)),
    )(x, thr_col)


# speedup vs baseline: 2.3103x; 1.7022x over previous
import functools

import jax
import jax.numpy as jnp
from jax import lax
from jax.experimental import pallas as pl
from jax.experimental.pallas import tpu as pltpu
from jax.experimental.pallas import tpu_sc as plsc

_K = 50
_ROWS = 128
_N = 32768
_NW = 32
_RPW = _ROWS // _NW
_G = 32
_NMAX = _N // _G
_INT_MIN = -(2**31)


def _skey(v):
    b = lax.bitcast_convert_type(v, jnp.int32)
    return jnp.where(b >= 0, b, b ^ jnp.int32(0x7FFFFFFF))


def _count_ge_blocks(ref, nblk, c):
    cvec = jnp.full((16,), c, dtype=jnp.int32)

    def body(g, cnt):
        for j in range(8):
            v = ref[pl.ds(g * 128 + j * 16, 16)]
            cnt = cnt + jnp.where(v >= cvec, jnp.int32(1), jnp.int32(0))
        return cnt

    cnt_v = lax.fori_loop(0, nblk, body, jnp.zeros((16,), jnp.int32))
    return jnp.sum(cnt_v)


def _search_blocks(ref, nblk, k):
    def it(i, p):
        c = p + (jnp.int32(1) << (jnp.int32(31) - i))
        cnt = _count_ge_blocks(ref, nblk, c)
        return jnp.where(cnt >= k, c, p)

    return lax.fori_loop(0, 32, it, jnp.int32(_INT_MIN))


def _key_to_f32(kvv):
    bv = jnp.where(kvv >= 0, kvv, kvv ^ jnp.int32(0x7FFFFFFF))
    return lax.bitcast_convert_type(bv, jnp.float32)


def _sc_row(row_v, maxes_v, cand_v):
    def groups(g, _):
        base = g * (_G * 16)
        acc = row_v[pl.ds(base, 16)]
        for j in range(1, _G):
            acc = jnp.maximum(acc, row_v[pl.ds(base + j * 16, 16)])
        maxes_v[pl.ds(g * 16, 16)] = _skey(acc)
        return 0

    lax.fori_loop(0, _NMAX // 16, groups, 0)

    t0 = _search_blocks(maxes_v, _NMAX // 128, jnp.int32(_K))
    t0f = _key_to_f32(jnp.full((16,), t0, dtype=jnp.int32))

    def collect(g, off):
        vbs, ms, pcs = [], [], []
        for j in range(8):
            v = row_v[pl.ds(g * 128 + j * 16, 16)]
            m = v >= t0f
            vbs.append(lax.bitcast_convert_type(v, jnp.int32))
            ms.append(m)
            pcs.append(plsc.all_reduce_population_count(m)[0])
        offs = [off]
        for j in range(8):
            offs.append(offs[j] + pcs[j])
        for j in range(8):
            plsc.store_compressed(cand_v.at[pl.ds(offs[j], 16)], vbs[j], mask=ms[j])
        return offs[8]

    off = lax.fori_loop(0, _N // 128, collect, jnp.int32(0))

    sent = jnp.full((16,), jnp.int32(-1))
    for j in range(8):
        cand_v[pl.ds(off + j * 16, 16)] = sent

    nblk = (off + jnp.int32(127)) // jnp.int32(128)

    def conv(g, _):
        for j in range(8):
            s = pl.ds(g * 128 + j * 16, 16)
            b = cand_v[s]
            cand_v[s] = jnp.where(b >= 0, b, b ^ jnp.int32(0x7FFFFFFF))
        return 0

    lax.fori_loop(0, nblk, conv, 0)
    return _search_blocks(cand_v, nblk, jnp.int32(_K))


def _sc_body(x_hbm, out_hbm, row_a, row_b, maxes_v, cand_v, thr_v, sem):
    wid = lax.axis_index("s") * 2 + lax.axis_index("c")
    thr_v[...] = jnp.zeros((16,), jnp.float32)
    lane = jax.lax.broadcasted_iota(jnp.int32, (16,), 0)

    bufs = [row_a, row_b]
    pltpu.sync_copy(x_hbm.at[wid * _RPW], bufs[0])
    for r in range(_RPW):
        cp = None
        if r + 1 < _RPW:
            cp = pltpu.async_copy(
                x_hbm.at[wid * _RPW + r + 1], bufs[(r + 1) % 2], sem
            )
        kv = _sc_row(bufs[r % 2], maxes_v, cand_v)
        tvv = _key_to_f32(jnp.full((16,), kv, dtype=jnp.int32))
        thr_v[...] = jnp.where(lane == r, tvv, thr_v[...])
        if cp is not None:
            cp.wait()

    pltpu.sync_copy(thr_v, out_hbm.at[wid])


_sc_thresholds = functools.partial(
    pl.kernel,
    out_type=jax.ShapeDtypeStruct((_NW, 16), jnp.float32),
    mesh=plsc.VectorSubcoreMesh(core_axis_name="c", subcore_axis_name="s"),
    compiler_params=pltpu.CompilerParams(needs_layout_passes=False),
    scratch_types=[
        pltpu.VMEM((_N,), jnp.float32),
        pltpu.VMEM((_N,), jnp.float32),
        pltpu.VMEM((_NMAX,), jnp.int32),
        pltpu.VMEM((_N + 128,), jnp.int32),
        pltpu.VMEM((16,), jnp.float32),
        pltpu.SemaphoreType.DMA,
    ],
)(_sc_body)


def _mask_kernel(x_ref, t_ref, o_ref):
    o_ref[...] = (x_ref[...] >= t_ref[...]).astype(jnp.float32)


@jax.jit
def kernel(x):
    m, n = x.shape
    thr = _sc_thresholds(x)
    thr_col = thr[:, :_RPW].reshape(m, 1)
    r = 16
    return pl.pallas_call(
        _mask_kernel,
        out_shape=jax.ShapeDtypeStruct((m, n), jnp.float32),
        grid=(m // r,),
        in_specs=[
            pl.BlockSpec((r, n), lambda i: (i, 0)),
            pl.BlockSpec((r, 1), lambda i: (i, 0)),
        ],
        out_specs=pl.BlockSpec((r, n), lambda i: (i, 0)),
    )(x, thr_col)
